# Initial kernel scaffold; baseline (speedup 1.0000x reference)
import jax
import jax.numpy as jnp
from jax.experimental import pallas as pl


def kernel(x, edge_index, mask_indices, W1, b1, W2, b2, Wp1, bp1, Wp2, bp2):
    raise NotImplementedError("write your pallas kernel here")


# speedup vs baseline: 6.0408x; 6.0408x over previous
import functools

import jax
import jax.numpy as jnp
from jax import lax
from jax.experimental import pallas as pl
from jax.experimental.pallas import tpu as pltpu
from jax.experimental.pallas import tpu_sc as plsc

N = 10000
E = 160000
D_FEAT = 256
HIDDEN = 512
EMB = 256
BETA = 0.9
T = 10
THRESH = 1.0
NUM_MASK = 1500

_NT = 16
_CK = 128
_NCH = 79
_EPT_PAD = _NCH * _CK
_W = 64
_ACC_R = 10016
_ZR = 632
_CR = 624
_TRASH = N
_NB = 4
_NGRP = 19

_f32 = jnp.float32
_i32 = jnp.int32


def _fill_const(ref, val, rows, cols):
    g = cols // 16

    def body(i, _):
        ref[i // g, pl.ds((i % g) * 16, 16)] = jnp.full((16,), val, _f32)
        return 0
    lax.fori_loop(0, rows * g, body, 0)


def _zero_acc_slice(acc, zeros_hbm, s):
    z0 = pl.multiple_of(jnp.minimum(s * _ZR, _ACC_R - _ZR), 8)
    pltpu.sync_copy(zeros_hbm, acc.at[pl.ds(z0, _ZR)])


def _copy_out_slice(acc, out_hbm, s, base):
    r0 = s * _CR
    pltpu.sync_copy(acc.at[pl.ds(pl.multiple_of(r0, 8), _CR)],
                    out_hbm.at[pl.ds(pl.multiple_of(base + r0, 8), _CR)])

    @pl.when(s == _NT - 1)
    def _tail():
        t0 = _NT * _CR
        pltpu.sync_copy(acc.at[pl.ds(t0, N - t0)],
                        out_hbm.at[pl.ds(pl.multiple_of(base + t0, 8), N - t0)])


def _make_spmm(items_per_sc):
    n_items = 2 * items_per_sc
    mesh = plsc.VectorSubcoreMesh(core_axis_name="c", subcore_axis_name="s")

    @functools.partial(
        pl.kernel, mesh=mesh,
        out_type=jax.ShapeDtypeStruct((n_items * N, _W), _f32),
        compiler_params=pltpu.CompilerParams(use_tc_tiling_on_sc=False),
        scratch_types=[
            pltpu.VMEM((_NCH, _CK), _i32),
            pltpu.VMEM((_NCH, _CK), _i32),
            pltpu.VMEM((_NB, _CK), _i32),
            pltpu.VMEM((_CK, _W), _f32),
            pltpu.VMEM((_CK, _W), _f32),
            pltpu.VMEM((_CK, _W), _f32),
            pltpu.VMEM((_CK, _W), _f32),
            pltpu.VMEM_SHARED((_ACC_R, _W), _f32),
            pltpu.SemaphoreType.DMA,
        ])
    def spmm(g_hbm, src_hbm, dst_hbm, zeros_hbm, out_hbm,
             src_v, dst_v, aidx, buf0, buf1, buf2, buf3, acc, sem):
        c = lax.axis_index("c")
        s = lax.axis_index("s")
        bufs = (buf0, buf1, buf2, buf3)

        pltpu.sync_copy(src_hbm.at[s], src_v)
        pltpu.sync_copy(dst_hbm.at[s], dst_v)

        def run_chunks(ch, nk, base):
            for k in range(nk):
                for u in range(_CK // 16):
                    aidx[k, pl.ds(u * 16, 16)] = (
                        src_v[ch + k, pl.ds(u * 16, 16)] + base)
            d = [pltpu.async_copy(g_hbm.at[aidx.at[k]], bufs[k], sem)
                 for k in range(nk)]
            for k in range(nk):
                d[k].wait()
            for k in range(nk):
                pltpu.sync_copy(bufs[k], acc.at[dst_v.at[ch + k]], add=True)

        def item_body(j, _):
            item = c * items_per_sc + j
            base = item * N

            plsc.subcore_barrier()
            _zero_acc_slice(acc, zeros_hbm, s)
            plsc.subcore_barrier()

            def group(i, _):
                run_chunks(_NB * i, _NB, base)
                return 0
            lax.fori_loop(0, _NGRP, group, 0)
            run_chunks(_NB * _NGRP, _NCH - _NB * _NGRP, base)

            plsc.subcore_barrier()
            _copy_out_slice(acc, out_hbm, s, base)
            return 0

        lax.fori_loop(0, items_per_sc, item_body, 0)

    return spmm


_spmm8 = _make_spmm(8)
_spmm40 = _make_spmm(40)


def _make_prep():
    mesh = plsc.VectorSubcoreMesh(core_axis_name="c", subcore_axis_name="s")

    @functools.partial(
        pl.kernel, mesh=mesh,
        out_type=jax.ShapeDtypeStruct((3 * N, _W), _f32),
        compiler_params=pltpu.CompilerParams(use_tc_tiling_on_sc=False),
        scratch_types=[
            pltpu.VMEM((40, _CK), _i32),
            pltpu.VMEM((1, _CK), _i32),
            pltpu.VMEM((_CK, _W), _f32),
            pltpu.VMEM_SHARED((_ACC_R, _W), _f32),
        ])
    def prep(dstp_hbm, maskp_hbm, zeros_hbm, out_hbm, dst_v, mask_v, ones, acc):
        c = lax.axis_index("c")
        s = lax.axis_index("s")
        pltpu.sync_copy(dstp_hbm.at[c].at[s], dst_v)
        pltpu.sync_copy(maskp_hbm.at[s], mask_v)
        _fill_const(ones, 1.0, _CK, _W)

        _zero_acc_slice(acc, zeros_hbm, s)
        plsc.subcore_barrier()

        def body(ch, _):
            pltpu.sync_copy(ones, acc.at[dst_v.at[ch]], add=True)
            return 0
        lax.fori_loop(0, 40, body, 0)
        plsc.subcore_barrier()
        _copy_out_slice(acc, out_hbm, s, c * N)

        @pl.when(c == 0)
        def _mask_phase():
            plsc.subcore_barrier()
            _zero_acc_slice(acc, zeros_hbm, s)
            plsc.subcore_barrier()
            pltpu.sync_copy(ones, acc.at[mask_v.at[0]], add=True)
            plsc.subcore_barrier()
            _copy_out_slice(acc, out_hbm, s, 2 * N)

    return prep


_prep = _make_prep()


def _dinv_of(degp_blk):
    deg = degp_blk[0, :, 0:1] + degp_blk[1, :, 0:1] + 1.0
    return lax.rsqrt(deg)


def _b1_body(x_ref, w1_ref, degp_ref, out_ref):
    dinv = _dinv_of(degp_ref)
    maskf = jnp.where(degp_ref[2, :, 0:1] > 0.0, 0.0, 1.0)
    h1 = jnp.dot(x_ref[...], w1_ref[...], preferred_element_type=_f32)
    g1 = h1 * dinv
    g1m = g1 * maskf
    for q in range(8):
        out_ref[q] = g1[:, q * _W:(q + 1) * _W]
        out_ref[8 + q] = g1m[:, q * _W:(q + 1) * _W]


def _b2_body(p1_ref, g1_ref, degp_ref, w2_ref, b1_ref, out_ref):
    dinv = _dinv_of(degp_ref)
    w2 = w2_ref[...]
    for e in range(2):
        tot = jnp.concatenate(
            [p1_ref[e * 8 + q] + g1_ref[e * 8 + q] for q in range(8)], axis=1)
        cur1 = dinv * tot + b1_ref[...]
        mem = jnp.zeros_like(cur1)
        for t in range(T):
            reset = (mem > THRESH).astype(_f32)
            mem = BETA * mem + cur1 - reset * THRESH
            spk = (mem > THRESH).astype(_f32)
            g2 = jnp.dot(spk, w2, preferred_element_type=_f32) * dinv
            i0 = (t * 2 + e) * 4
            for q in range(4):
                out_ref[i0 + q] = g2[:, q * _W:(q + 1) * _W]


def _b3_body(p2_ref, g2_ref, degp_ref, b2_ref, wp1_ref, bp1_ref, wp2_ref,
             bp2_ref, pred_ref, tgt_ref):
    dinv = _dinv_of(degp_ref)
    embs = []
    for e in range(2):
        mem = jnp.zeros((p2_ref.shape[1], EMB), _f32)
        ssum = jnp.zeros_like(mem)
        for t in range(T):
            i0 = (t * 2 + e) * 4
            tot = jnp.concatenate(
                [p2_ref[i0 + q] + g2_ref[i0 + q] for q in range(4)], axis=1)
            cur2 = dinv * tot + b2_ref[...]
            reset = (mem > THRESH).astype(_f32)
            mem = BETA * mem + cur2 - reset * THRESH
            ssum = ssum + (mem > THRESH).astype(_f32)
        embs.append(ssum / T)
    tgt_ref[...] = embs[0]
    ctx = embs[1]
    hh = jnp.maximum(
        jnp.dot(ctx, wp1_ref[...], preferred_element_type=_f32) + bp1_ref[...],
        0.0)
    pred_ref[...] = (jnp.dot(hh, wp2_ref[...], preferred_element_type=_f32)
                     + bp2_ref[...])


def _row_spec(r, shape):
    if len(shape) == 2:
        return pl.BlockSpec((r, shape[1]), lambda i: (i, 0))
    return pl.BlockSpec((shape[0], r, shape[2]), lambda i: (0, i, 0))


def _full_spec(shape):
    nd = len(shape)
    return pl.BlockSpec(shape, (lambda i: (0,) * nd))


def _b1(x, w1, degp):
    r = 400
    return pl.pallas_call(
        _b1_body,
        grid=(N // r,),
        in_specs=[_row_spec(r, (N, D_FEAT)), _full_spec((D_FEAT, HIDDEN)),
                  _row_spec(r, (3, N, _W))],
        out_specs=_row_spec(r, (16, N, _W)),
        out_shape=jax.ShapeDtypeStruct((16, N, _W), _f32),
    )(x, w1, degp)


def _b2(p1, g1, degp, w2, b1r):
    r = 400
    return pl.pallas_call(
        _b2_body,
        grid=(N // r,),
        in_specs=[_row_spec(r, (16, N, _W)), _row_spec(r, (16, N, _W)),
                  _row_spec(r, (3, N, _W)), _full_spec((HIDDEN, EMB)),
                  _full_spec((1, HIDDEN))],
        out_specs=_row_spec(r, (80, N, _W)),
        out_shape=jax.ShapeDtypeStruct((80, N, _W), _f32),
    )(p1, g1, degp, w2, b1r)


def _b3(p2, g2, degp, b2r, wp1, bp1r, wp2, bp2r):
    r = 200
    return pl.pallas_call(
        _b3_body,
        grid=(N // r,),
        in_specs=[_row_spec(r, (80, N, _W)), _row_spec(r, (80, N, _W)),
                  _row_spec(r, (3, N, _W)), _full_spec((1, EMB)),
                  _full_spec((EMB, HIDDEN)), _full_spec((1, HIDDEN)),
                  _full_spec((HIDDEN, EMB)), _full_spec((1, EMB))],
        out_specs=[_row_spec(r, (N, EMB)), _row_spec(r, (N, EMB))],
        out_shape=[jax.ShapeDtypeStruct((N, EMB), _f32),
                   jax.ShapeDtypeStruct((N, EMB), _f32)],
    )(p2, g2, degp, b2r, wp1, bp1r, wp2, bp2r)


def kernel(x, edge_index, mask_indices, W1, b1, W2, b2, Wp1, bp1, Wp2, bp2):
    src = edge_index[0].astype(_i32)
    dst = edge_index[1].astype(_i32)

    pad = _NT * _EPT_PAD - E
    srcp = jnp.pad(src, (0, pad)).reshape(_NT, _NCH, _CK)
    dstp = jnp.pad(dst, (0, pad), constant_values=_TRASH).reshape(
        _NT, _NCH, _CK)
    dstp2 = jnp.pad(dst.reshape(2, _NT, 5000), ((0, 0), (0, 0), (0, 120)),
                    constant_values=_TRASH).reshape(2, _NT, 40, _CK)
    maskp = jnp.pad(mask_indices.astype(_i32), (0, _NT * _CK - NUM_MASK),
                    constant_values=_TRASH).reshape(_NT, 1, _CK)
    zeros = jnp.zeros((_ZR, _W), _f32)

    degp = _prep(dstp2, maskp, zeros).reshape(3, N, _W)
    g1 = _b1(x, W1, degp)
    p1 = _spmm8(g1.reshape(16 * N, _W), srcp, dstp, zeros).reshape(16, N, _W)
    g2 = _b2(p1, g1, degp, W2, b1.reshape(1, HIDDEN))
    p2 = _spmm40(g2.reshape(80 * N, _W), srcp, dstp, zeros).reshape(80, N, _W)
    pred, tgt = _b3(p2, g2, degp, b2.reshape(1, EMB), Wp1,
                    bp1.reshape(1, HIDDEN), Wp2, bp2.reshape(1, EMB))
    return pred, tgt
